# baseline (speedup 1.0000x reference)
import functools

import jax
import jax.numpy as jnp
from jax import lax
from jax.experimental import pallas as pl
from jax.experimental.pallas import tpu as pltpu
from jax.experimental.pallas import tpu_sc as plsc

N = 10000
E = 320000
D = 128
H = 64
G = 64
EPS = 1e-5

NPAD = 10240
CH = 128
NCHUNKS = E // CH
BASE_CHUNKS = NCHUNKS // 32
EXTRA_TILES = NCHUNKS - 32 * BASE_CHUNKS
ROWS_PER_TILE = NPAD // 16
NB = 3
NGRP = BASE_CHUNKS // NB

BN = 1000
BN2 = 1024
GRID2 = NPAD // BN2

_mesh = plsc.VectorSubcoreMesh(core_axis_name="c", subcore_axis_name="s")


def _tile_range(gid):
    start = gid * BASE_CHUNKS + jnp.minimum(gid, EXTRA_TILES)
    has_extra = gid < EXTRA_TILES
    return start, has_extra


@functools.partial(
    pl.kernel,
    out_type=jax.ShapeDtypeStruct((2 * NPAD,), jnp.float32),
    mesh=_mesh,
    compiler_params=pltpu.CompilerParams(use_tc_tiling_on_sc=False),
    scratch_types=[
        pltpu.VMEM((BASE_CHUNKS + 1, 1, CH), jnp.int32),
        pltpu.VMEM((CH,), jnp.float32),
        pltpu.VMEM((ROWS_PER_TILE,), jnp.float32),
        pltpu.VMEM_SHARED((NPAD,), jnp.float32),
    ],
)
def _deg_kernel(dst_hbm, out_hbm, didx, ones_v, zbuf_v, cnt_sp):
    c = lax.axis_index("c")
    s = lax.axis_index("s")
    gid = c * 16 + s
    start, has_extra = _tile_range(gid)

    def fill(i, _):
        zbuf_v[pl.ds(i * 16, 16)] = jnp.zeros((16,), jnp.float32)
        return 0

    lax.fori_loop(0, ROWS_PER_TILE // 16, fill, 0)

    def fill1(i, _):
        ones_v[pl.ds(i * 16, 16)] = jnp.ones((16,), jnp.float32)
        return 0

    lax.fori_loop(0, CH // 16, fill1, 0)

    pltpu.sync_copy(dst_hbm.at[pl.ds(start, BASE_CHUNKS)],
                    didx.at[pl.ds(0, BASE_CHUNKS)])

    @pl.when(has_extra)
    def _():
        pltpu.sync_copy(dst_hbm.at[pl.ds(start + BASE_CHUNKS, 1)],
                        didx.at[pl.ds(BASE_CHUNKS, 1)])

    pltpu.sync_copy(zbuf_v, cnt_sp.at[pl.ds(s * ROWS_PER_TILE, ROWS_PER_TILE)])
    plsc.subcore_barrier()

    nch = BASE_CHUNKS + has_extra.astype(jnp.int32)

    def body(k, _):
        pltpu.sync_copy(ones_v, cnt_sp.at[didx.at[k, 0]], add=True)
        return 0

    lax.fori_loop(0, nch, body, 0)
    plsc.subcore_barrier()
    pltpu.sync_copy(
        cnt_sp.at[pl.ds(s * ROWS_PER_TILE, ROWS_PER_TILE)],
        out_hbm.at[pl.ds(c * NPAD + s * ROWS_PER_TILE, ROWS_PER_TILE)],
    )


@functools.partial(
    pl.kernel,
    out_type=jax.ShapeDtypeStruct((2 * NPAD, H), jnp.float32),
    mesh=_mesh,
    compiler_params=pltpu.CompilerParams(use_tc_tiling_on_sc=False),
    scratch_types=[
        pltpu.VMEM((BASE_CHUNKS + 1, 1, CH), jnp.int32),
        pltpu.VMEM((BASE_CHUNKS + 1, 1, CH), jnp.int32),
        pltpu.VMEM((NB, CH, H), jnp.float32),
        pltpu.VMEM((NB, CH, H), jnp.float32),
        pltpu.VMEM_SHARED((NPAD, H), jnp.float32),
        pltpu.SemaphoreType.DMA,
        pltpu.SemaphoreType.DMA,
        pltpu.SemaphoreType.DMA,
    ],
)
def _agg_kernel(hp_hbm, src_hbm, dst_hbm, out_hbm, sidx, didx, rows_a, rows_b,
                acc_sp, gsem, ssem_a, ssem_b):
    c = lax.axis_index("c")
    s = lax.axis_index("s")
    gid = c * 16 + s
    start, has_extra = _tile_range(gid)

    def fill(t, _):
        rows_a[0, t // 4, pl.ds((t % 4) * 16, 16)] = jnp.zeros((16,), jnp.float32)
        return 0

    lax.fori_loop(0, CH * (H // 16), fill, 0)

    def zc(k, _):
        pltpu.sync_copy(rows_a.at[0],
                        acc_sp.at[pl.ds(s * ROWS_PER_TILE + k * CH, CH), :])
        return 0

    lax.fori_loop(0, ROWS_PER_TILE // CH, zc, 0)

    pltpu.sync_copy(src_hbm.at[pl.ds(start, BASE_CHUNKS)],
                    sidx.at[pl.ds(0, BASE_CHUNKS)])
    pltpu.sync_copy(dst_hbm.at[pl.ds(start, BASE_CHUNKS)],
                    didx.at[pl.ds(0, BASE_CHUNKS)])

    @pl.when(has_extra)
    def _():
        pltpu.sync_copy(src_hbm.at[pl.ds(start + BASE_CHUNKS, 1)],
                        sidx.at[pl.ds(BASE_CHUNKS, 1)])
        pltpu.sync_copy(dst_hbm.at[pl.ds(start + BASE_CHUNKS, 1)],
                        didx.at[pl.ds(BASE_CHUNKS, 1)])

    plsc.subcore_barrier()

    def fire_g(k, rset):
        for b in range(NB):
            pltpu.async_copy(hp_hbm.at[sidx.at[k + b, 0]], rset.at[b], gsem)

    def wait_g(k, rset):
        for b in range(NB):
            pltpu.make_async_copy(hp_hbm.at[sidx.at[k + b, 0]], rset.at[b],
                                  gsem).wait()

    def fire_s(k, rset, ssem):
        for b in range(NB):
            pltpu.async_copy(rset.at[b], acc_sp.at[didx.at[k + b, 0]], ssem,
                             add=True)

    def wait_s(k, rset, ssem):
        for b in range(NB):
            pltpu.make_async_copy(rset.at[b], acc_sp.at[didx.at[k + b, 0]],
                                  ssem).wait()

    fire_g(0, rows_a)

    def pair(i, _):
        ka = (2 * i) * NB
        kb = ka + NB
        wait_g(ka, rows_a)
        fire_s(ka, rows_a, ssem_a)

        @pl.when(i > 0)
        def _():
            wait_s(ka - NB, rows_b, ssem_b)

        fire_g(kb, rows_b)
        wait_g(kb, rows_b)
        fire_s(kb, rows_b, ssem_b)
        wait_s(ka, rows_a, ssem_a)
        fire_g(kb + NB, rows_a)
        return 0

    lax.fori_loop(0, NGRP // 2 - 1, pair, 0)

    ka = (NGRP - 2) * NB
    kb = (NGRP - 1) * NB
    wait_g(ka, rows_a)
    fire_s(ka, rows_a, ssem_a)
    wait_s(ka - NB, rows_b, ssem_b)
    fire_g(kb, rows_b)
    wait_g(kb, rows_b)
    fire_s(kb, rows_b, ssem_b)
    wait_s(ka, rows_a, ssem_a)

    @pl.when(has_extra)
    def _():
        pltpu.async_copy(hp_hbm.at[sidx.at[BASE_CHUNKS, 0]], rows_a.at[0], gsem).wait()
        pltpu.sync_copy(rows_a.at[0], acc_sp.at[didx.at[BASE_CHUNKS, 0]], add=True)

    wait_s(kb, rows_b, ssem_b)
    plsc.subcore_barrier()
    pltpu.sync_copy(
        acc_sp.at[pl.ds(s * ROWS_PER_TILE, ROWS_PER_TILE), :],
        out_hbm.at[pl.ds(c * NPAD + s * ROWS_PER_TILE, ROWS_PER_TILE), :],
    )


def _mm1_body(cnt_ref, x_ref, w1_ref, hp_ref):
    cnt2 = cnt_ref[...]
    deg = cnt2[0] + cnt2[1] + 1.0
    hp_ref[...] = jnp.dot(x_ref[...], w1_ref[...],
                          preferred_element_type=jnp.float32) * lax.rsqrt(deg)


def _dense2_body(parts_ref, h1p_ref, cnt_ref, b1_ref, g1_ref, be1_ref, w2_ref,
                 h2p_ref):
    p = parts_ref[...]
    agg = p[0] + p[1]
    cnt2 = cnt_ref[...]
    deg = cnt2[0] + cnt2[1] + 1.0
    dinv = lax.rsqrt(deg)
    z = dinv * (agg + h1p_ref[...]) + b1_ref[...]
    z = z * (g1_ref[...] / jnp.sqrt(1.0 + EPS)) + be1_ref[...]
    z = jnp.maximum(z, 0.0)
    h2p_ref[...] = jnp.dot(z, w2_ref[...],
                           preferred_element_type=jnp.float32) * dinv


def _pool_body(parts_ref, h2p_ref, cnt_ref, b2_ref, g2_ref, be2_ref, batch_ref,
               out_ref, sums, cnts):
    i = pl.program_id(0)

    @pl.when(i == 0)
    def _():
        sums[...] = jnp.zeros_like(sums)
        cnts[...] = jnp.zeros_like(cnts)

    p = parts_ref[...]
    agg = p[0] + p[1]
    cnt2 = cnt_ref[...]
    deg = cnt2[0] + cnt2[1] + 1.0
    dinv = lax.rsqrt(deg)
    z = dinv * (agg + h2p_ref[...]) + b2_ref[...]
    z = z * (g2_ref[...] / jnp.sqrt(1.0 + EPS)) + be2_ref[...]
    z = jnp.maximum(z, 0.0)

    b = batch_ref[...]
    z = jnp.where(b < G, z, 0.0)
    onehot = (b == lax.broadcasted_iota(jnp.int32, (BN2, G), 1)).astype(jnp.float32)
    sums[...] += lax.dot_general(onehot, z, (((0,), (0,)), ((), ())),
                                 preferred_element_type=jnp.float32)
    cnts[...] += lax.dot_general(onehot, jnp.ones((BN2, 1), jnp.float32),
                                 (((0,), (0,)), ((), ())),
                                 preferred_element_type=jnp.float32)
    out_ref[...] = sums[...] / jnp.maximum(cnts[...], 1.0)


def kernel(x, edge_index, batch, W1, b1, g1, beta1, W2, b2, g2, beta2):
    src3 = edge_index[0].astype(jnp.int32).reshape(NCHUNKS, 1, CH)
    dst3 = edge_index[1].astype(jnp.int32).reshape(NCHUNKS, 1, CH)
    batch_p = jnp.pad(batch.astype(jnp.int32), (0, NPAD - N),
                      constant_values=G).reshape(NPAD, 1)

    cnt = _deg_kernel(dst3).reshape(2, NPAD, 1)

    h1p = pl.pallas_call(
        _mm1_body,
        grid=(N // BN,),
        in_specs=[pl.BlockSpec((2, BN, 1), lambda i: (0, i, 0)),
                  pl.BlockSpec((BN, D), lambda i: (i, 0)),
                  pl.BlockSpec((D, H), lambda i: (0, 0))],
        out_specs=pl.BlockSpec((BN, H), lambda i: (i, 0)),
        out_shape=jax.ShapeDtypeStruct((NPAD, H), jnp.float32),
    )(cnt, x, W1)

    cnt_spec = pl.BlockSpec((2, BN2, 1), lambda i: (0, i, 0))
    row_spec = pl.BlockSpec((BN2, H), lambda i: (i, 0))
    vec_spec = pl.BlockSpec((1, H), lambda i: (0, 0))
    parts_spec = pl.BlockSpec((2, BN2, H), lambda i: (0, i, 0))

    agg1 = _agg_kernel(h1p, src3, dst3).reshape(2, NPAD, H)

    h2p = pl.pallas_call(
        _dense2_body,
        grid=(GRID2,),
        in_specs=[parts_spec, row_spec, cnt_spec, vec_spec, vec_spec, vec_spec,
                  pl.BlockSpec((H, H), lambda i: (0, 0))],
        out_specs=row_spec,
        out_shape=jax.ShapeDtypeStruct((NPAD, H), jnp.float32),
    )(agg1, h1p, cnt, b1.reshape(1, H), g1.reshape(1, H), beta1.reshape(1, H),
      W2)

    agg2 = _agg_kernel(h2p, src3, dst3).reshape(2, NPAD, H)

    emb = pl.pallas_call(
        _pool_body,
        grid=(GRID2,),
        in_specs=[parts_spec, row_spec, cnt_spec, vec_spec, vec_spec, vec_spec,
                  pl.BlockSpec((BN2, 1), lambda i: (i, 0))],
        out_specs=pl.BlockSpec((G, H), lambda i: (0, 0)),
        out_shape=jax.ShapeDtypeStruct((G, H), jnp.float32),
        scratch_shapes=[pltpu.VMEM((G, H), jnp.float32),
                        pltpu.VMEM((G, 1), jnp.float32)],
    )(agg2, h2p, cnt, b2.reshape(1, H), g2.reshape(1, H), beta2.reshape(1, H),
      batch_p)
    return emb


# speedup vs baseline: 1.0001x; 1.0001x over previous
import functools

import jax
import jax.numpy as jnp
from jax import lax
from jax.experimental import pallas as pl
from jax.experimental.pallas import tpu as pltpu
from jax.experimental.pallas import tpu_sc as plsc

N = 10000
E = 320000
D = 128
H = 64
G = 64
EPS = 1e-5

NPAD = 10240
CH = 128
NCHUNKS = E // CH
BASE_CHUNKS = NCHUNKS // 32
EXTRA_TILES = NCHUNKS - 32 * BASE_CHUNKS
ROWS_PER_TILE = NPAD // 16
NB = 3
NGRP = BASE_CHUNKS // NB

BN = 1000
BN2 = 1024
GRID2 = NPAD // BN2

_mesh = plsc.VectorSubcoreMesh(core_axis_name="c", subcore_axis_name="s")


def _tile_range(gid):
    start = gid * BASE_CHUNKS + jnp.minimum(gid, EXTRA_TILES)
    has_extra = gid < EXTRA_TILES
    return start, has_extra


@functools.partial(
    pl.kernel,
    out_type=jax.ShapeDtypeStruct((2 * NPAD,), jnp.float32),
    mesh=_mesh,
    compiler_params=pltpu.CompilerParams(use_tc_tiling_on_sc=False),
    scratch_types=[
        pltpu.VMEM((BASE_CHUNKS + 1, 1, CH), jnp.int32),
        pltpu.VMEM((CH,), jnp.float32),
        pltpu.VMEM((ROWS_PER_TILE,), jnp.float32),
        pltpu.VMEM_SHARED((NPAD,), jnp.float32),
    ],
)
def _deg_kernel(dst_hbm, out_hbm, didx, ones_v, zbuf_v, cnt_sp):
    c = lax.axis_index("c")
    s = lax.axis_index("s")
    gid = c * 16 + s
    start, has_extra = _tile_range(gid)

    def fill(i, _):
        zbuf_v[pl.ds(i * 16, 16)] = jnp.zeros((16,), jnp.float32)
        return 0

    lax.fori_loop(0, ROWS_PER_TILE // 16, fill, 0)

    def fill1(i, _):
        ones_v[pl.ds(i * 16, 16)] = jnp.ones((16,), jnp.float32)
        return 0

    lax.fori_loop(0, CH // 16, fill1, 0)

    pltpu.sync_copy(dst_hbm.at[pl.ds(start, BASE_CHUNKS)],
                    didx.at[pl.ds(0, BASE_CHUNKS)])

    @pl.when(has_extra)
    def _():
        pltpu.sync_copy(dst_hbm.at[pl.ds(start + BASE_CHUNKS, 1)],
                        didx.at[pl.ds(BASE_CHUNKS, 1)])

    pltpu.sync_copy(zbuf_v, cnt_sp.at[pl.ds(s * ROWS_PER_TILE, ROWS_PER_TILE)])
    plsc.subcore_barrier()

    nch = BASE_CHUNKS + has_extra.astype(jnp.int32)

    def body(k, _):
        pltpu.sync_copy(ones_v, cnt_sp.at[didx.at[k, 0]], add=True)
        return 0

    lax.fori_loop(0, nch, body, 0)
    plsc.subcore_barrier()
    pltpu.sync_copy(
        cnt_sp.at[pl.ds(s * ROWS_PER_TILE, ROWS_PER_TILE)],
        out_hbm.at[pl.ds(c * NPAD + s * ROWS_PER_TILE, ROWS_PER_TILE)],
    )


@functools.partial(
    pl.kernel,
    out_type=jax.ShapeDtypeStruct((2, NPAD, H), jnp.float32),
    mesh=_mesh,
    compiler_params=pltpu.CompilerParams(use_tc_tiling_on_sc=False),
    scratch_types=[
        pltpu.VMEM((BASE_CHUNKS + 1, 1, CH), jnp.int32),
        pltpu.VMEM((BASE_CHUNKS + 1, 1, CH), jnp.int32),
        pltpu.VMEM((NB, CH, H), jnp.float32),
        pltpu.VMEM((NB, CH, H), jnp.float32),
        pltpu.VMEM_SHARED((NPAD, H), jnp.float32),
        pltpu.SemaphoreType.DMA,
        pltpu.SemaphoreType.DMA,
        pltpu.SemaphoreType.DMA,
    ],
)
def _agg_kernel(hp_hbm, src_hbm, dst_hbm, out_hbm, sidx, didx, rows_a, rows_b,
                acc_sp, gsem, ssem_a, ssem_b):
    c = lax.axis_index("c")
    s = lax.axis_index("s")
    gid = c * 16 + s
    start, has_extra = _tile_range(gid)

    def fill(t, _):
        rows_a[0, t // 4, pl.ds((t % 4) * 16, 16)] = jnp.zeros((16,), jnp.float32)
        return 0

    lax.fori_loop(0, CH * (H // 16), fill, 0)

    def zc(k, _):
        pltpu.sync_copy(rows_a.at[0],
                        acc_sp.at[pl.ds(s * ROWS_PER_TILE + k * CH, CH), :])
        return 0

    lax.fori_loop(0, ROWS_PER_TILE // CH, zc, 0)

    pltpu.sync_copy(src_hbm.at[pl.ds(start, BASE_CHUNKS)],
                    sidx.at[pl.ds(0, BASE_CHUNKS)])
    pltpu.sync_copy(dst_hbm.at[pl.ds(start, BASE_CHUNKS)],
                    didx.at[pl.ds(0, BASE_CHUNKS)])

    @pl.when(has_extra)
    def _():
        pltpu.sync_copy(src_hbm.at[pl.ds(start + BASE_CHUNKS, 1)],
                        sidx.at[pl.ds(BASE_CHUNKS, 1)])
        pltpu.sync_copy(dst_hbm.at[pl.ds(start + BASE_CHUNKS, 1)],
                        didx.at[pl.ds(BASE_CHUNKS, 1)])

    plsc.subcore_barrier()

    def fire_g(k, rset):
        for b in range(NB):
            pltpu.async_copy(hp_hbm.at[sidx.at[k + b, 0]], rset.at[b], gsem)

    def wait_g(k, rset):
        for b in range(NB):
            pltpu.make_async_copy(hp_hbm.at[sidx.at[k + b, 0]], rset.at[b],
                                  gsem).wait()

    def fire_s(k, rset, ssem):
        for b in range(NB):
            pltpu.async_copy(rset.at[b], acc_sp.at[didx.at[k + b, 0]], ssem,
                             add=True)

    def wait_s(k, rset, ssem):
        for b in range(NB):
            pltpu.make_async_copy(rset.at[b], acc_sp.at[didx.at[k + b, 0]],
                                  ssem).wait()

    fire_g(0, rows_a)

    def pair(i, _):
        ka = (2 * i) * NB
        kb = ka + NB
        wait_g(ka, rows_a)
        fire_s(ka, rows_a, ssem_a)

        @pl.when(i > 0)
        def _():
            wait_s(ka - NB, rows_b, ssem_b)

        fire_g(kb, rows_b)
        wait_g(kb, rows_b)
        fire_s(kb, rows_b, ssem_b)
        wait_s(ka, rows_a, ssem_a)
        fire_g(kb + NB, rows_a)
        return 0

    lax.fori_loop(0, NGRP // 2 - 1, pair, 0)

    ka = (NGRP - 2) * NB
    kb = (NGRP - 1) * NB
    wait_g(ka, rows_a)
    fire_s(ka, rows_a, ssem_a)
    wait_s(ka - NB, rows_b, ssem_b)
    fire_g(kb, rows_b)
    wait_g(kb, rows_b)
    fire_s(kb, rows_b, ssem_b)
    wait_s(ka, rows_a, ssem_a)

    @pl.when(has_extra)
    def _():
        pltpu.async_copy(hp_hbm.at[sidx.at[BASE_CHUNKS, 0]], rows_a.at[0], gsem).wait()
        pltpu.sync_copy(rows_a.at[0], acc_sp.at[didx.at[BASE_CHUNKS, 0]], add=True)

    wait_s(kb, rows_b, ssem_b)
    plsc.subcore_barrier()
    pltpu.sync_copy(
        acc_sp.at[pl.ds(s * ROWS_PER_TILE, ROWS_PER_TILE), :],
        out_hbm.at[c, pl.ds(s * ROWS_PER_TILE, ROWS_PER_TILE), :],
    )


def _mm1_body(cnt_ref, x_ref, w1_ref, hp_ref):
    cnt2 = cnt_ref[...]
    deg = cnt2[0] + cnt2[1] + 1.0
    hp_ref[...] = jnp.dot(x_ref[...], w1_ref[...],
                          preferred_element_type=jnp.float32) * lax.rsqrt(deg)


def _dense2_body(parts_ref, h1p_ref, cnt_ref, b1_ref, g1_ref, be1_ref, w2_ref,
                 h2p_ref):
    p = parts_ref[...]
    agg = p[0] + p[1]
    cnt2 = cnt_ref[...]
    deg = cnt2[0] + cnt2[1] + 1.0
    dinv = lax.rsqrt(deg)
    z = dinv * (agg + h1p_ref[...]) + b1_ref[...]
    z = z * (g1_ref[...] / jnp.sqrt(1.0 + EPS)) + be1_ref[...]
    z = jnp.maximum(z, 0.0)
    h2p_ref[...] = jnp.dot(z, w2_ref[...],
                           preferred_element_type=jnp.float32) * dinv


def _pool_body(parts_ref, h2p_ref, cnt_ref, b2_ref, g2_ref, be2_ref, batch_ref,
               out_ref, sums, cnts):
    i = pl.program_id(0)

    @pl.when(i == 0)
    def _():
        sums[...] = jnp.zeros_like(sums)
        cnts[...] = jnp.zeros_like(cnts)

    p = parts_ref[...]
    agg = p[0] + p[1]
    cnt2 = cnt_ref[...]
    deg = cnt2[0] + cnt2[1] + 1.0
    dinv = lax.rsqrt(deg)
    z = dinv * (agg + h2p_ref[...]) + b2_ref[...]
    z = z * (g2_ref[...] / jnp.sqrt(1.0 + EPS)) + be2_ref[...]
    z = jnp.maximum(z, 0.0)

    b = batch_ref[...]
    z = jnp.where(b < G, z, 0.0)
    onehot = (b == lax.broadcasted_iota(jnp.int32, (BN2, G), 1)).astype(jnp.float32)
    sums[...] += lax.dot_general(onehot, z, (((0,), (0,)), ((), ())),
                                 preferred_element_type=jnp.float32)
    cnts[...] += lax.dot_general(onehot, jnp.ones((BN2, 1), jnp.float32),
                                 (((0,), (0,)), ((), ())),
                                 preferred_element_type=jnp.float32)
    out_ref[...] = sums[...] / jnp.maximum(cnts[...], 1.0)


def kernel(x, edge_index, batch, W1, b1, g1, beta1, W2, b2, g2, beta2):
    src3 = edge_index[0].astype(jnp.int32).reshape(NCHUNKS, 1, CH)
    dst3 = edge_index[1].astype(jnp.int32).reshape(NCHUNKS, 1, CH)
    batch_p = jnp.pad(batch.astype(jnp.int32), (0, NPAD - N),
                      constant_values=G).reshape(NPAD, 1)

    cnt = _deg_kernel(dst3).reshape(2, NPAD, 1)

    h1p = pl.pallas_call(
        _mm1_body,
        grid=(N // BN,),
        in_specs=[pl.BlockSpec((2, BN, 1), lambda i: (0, i, 0)),
                  pl.BlockSpec((BN, D), lambda i: (i, 0)),
                  pl.BlockSpec((D, H), lambda i: (0, 0))],
        out_specs=pl.BlockSpec((BN, H), lambda i: (i, 0)),
        out_shape=jax.ShapeDtypeStruct((NPAD, H), jnp.float32),
    )(cnt, x, W1)

    cnt_spec = pl.BlockSpec((2, BN2, 1), lambda i: (0, i, 0))
    row_spec = pl.BlockSpec((BN2, H), lambda i: (i, 0))
    vec_spec = pl.BlockSpec((1, H), lambda i: (0, 0))
    parts_spec = pl.BlockSpec((2, BN2, H), lambda i: (0, i, 0))

    agg1 = _agg_kernel(h1p, src3, dst3)

    h2p = pl.pallas_call(
        _dense2_body,
        grid=(GRID2,),
        in_specs=[parts_spec, row_spec, cnt_spec, vec_spec, vec_spec, vec_spec,
                  pl.BlockSpec((H, H), lambda i: (0, 0))],
        out_specs=row_spec,
        out_shape=jax.ShapeDtypeStruct((NPAD, H), jnp.float32),
    )(agg1, h1p, cnt, b1.reshape(1, H), g1.reshape(1, H), beta1.reshape(1, H),
      W2)

    agg2 = _agg_kernel(h2p, src3, dst3)

    emb = pl.pallas_call(
        _pool_body,
        grid=(GRID2,),
        in_specs=[parts_spec, row_spec, cnt_spec, vec_spec, vec_spec, vec_spec,
                  pl.BlockSpec((BN2, 1), lambda i: (i, 0))],
        out_specs=pl.BlockSpec((G, H), lambda i: (0, 0)),
        out_shape=jax.ShapeDtypeStruct((G, H), jnp.float32),
        scratch_shapes=[pltpu.VMEM((G, H), jnp.float32),
                        pltpu.VMEM((G, 1), jnp.float32)],
    )(agg2, h2p, cnt, b2.reshape(1, H), g2.reshape(1, H), beta2.reshape(1, H),
      batch_p)
    return emb


# speedup vs baseline: 1.0565x; 1.0564x over previous
import functools

import jax
import jax.numpy as jnp
from jax import lax
from jax.experimental import pallas as pl
from jax.experimental.pallas import tpu as pltpu
from jax.experimental.pallas import tpu_sc as plsc

N = 10000
E = 320000
D = 128
H = 64
G = 64
EPS = 1e-5

NPAD = 10240
CH = 128
NCHUNKS = E // CH
BASE_CHUNKS = NCHUNKS // 32
EXTRA_TILES = NCHUNKS - 32 * BASE_CHUNKS
ROWS_PER_TILE = NPAD // 16
NB = 3
NGRP = BASE_CHUNKS // NB

BN2 = 1024
GRID2 = NPAD // BN2

_mesh = plsc.VectorSubcoreMesh(core_axis_name="c", subcore_axis_name="s")


def _tile_range(gid):
    start = gid * BASE_CHUNKS + jnp.minimum(gid, EXTRA_TILES)
    has_extra = gid < EXTRA_TILES
    return start, has_extra


@functools.partial(
    pl.kernel,
    out_type=jax.ShapeDtypeStruct((2, NPAD), jnp.float32),
    mesh=_mesh,
    compiler_params=pltpu.CompilerParams(use_tc_tiling_on_sc=False),
    scratch_types=[
        pltpu.VMEM((BASE_CHUNKS + 1, CH), jnp.int32),
        pltpu.VMEM((CH,), jnp.float32),
        pltpu.VMEM((ROWS_PER_TILE,), jnp.float32),
        pltpu.VMEM_SHARED((NPAD,), jnp.float32),
    ],
)
def _deg_kernel(dst_hbm, out_hbm, didx, ones_v, zbuf_v, cnt_sp):
    c = lax.axis_index("c")
    s = lax.axis_index("s")
    gid = c * 16 + s
    start, has_extra = _tile_range(gid)

    def fill(i, _):
        zbuf_v[pl.ds(i * 16, 16)] = jnp.zeros((16,), jnp.float32)
        return 0

    lax.fori_loop(0, ROWS_PER_TILE // 16, fill, 0)

    def fill1(i, _):
        ones_v[pl.ds(i * 16, 16)] = jnp.ones((16,), jnp.float32)
        return 0

    lax.fori_loop(0, CH // 16, fill1, 0)

    pltpu.sync_copy(dst_hbm.at[pl.ds(start, BASE_CHUNKS)],
                    didx.at[pl.ds(0, BASE_CHUNKS)])

    @pl.when(has_extra)
    def _():
        pltpu.sync_copy(dst_hbm.at[pl.ds(start + BASE_CHUNKS, 1)],
                        didx.at[pl.ds(BASE_CHUNKS, 1)])

    pltpu.sync_copy(zbuf_v, cnt_sp.at[pl.ds(s * ROWS_PER_TILE, ROWS_PER_TILE)])
    plsc.subcore_barrier()

    nch = BASE_CHUNKS + has_extra.astype(jnp.int32)

    def body(k, _):
        pltpu.sync_copy(ones_v, cnt_sp.at[didx.at[k]], add=True)
        return 0

    lax.fori_loop(0, nch, body, 0)
    plsc.subcore_barrier()
    pltpu.sync_copy(
        cnt_sp.at[pl.ds(s * ROWS_PER_TILE, ROWS_PER_TILE)],
        out_hbm.at[c, pl.ds(s * ROWS_PER_TILE, ROWS_PER_TILE)],
    )


@functools.partial(
    pl.kernel,
    out_type=jax.ShapeDtypeStruct((2, NPAD, H), jnp.float32),
    mesh=_mesh,
    compiler_params=pltpu.CompilerParams(use_tc_tiling_on_sc=False),
    scratch_types=[
        pltpu.VMEM((BASE_CHUNKS + 1, CH), jnp.int32),
        pltpu.VMEM((BASE_CHUNKS + 1, CH), jnp.int32),
        pltpu.VMEM((NB, CH, H), jnp.float32),
        pltpu.VMEM((NB, CH, H), jnp.float32),
        pltpu.VMEM_SHARED((NPAD, H), jnp.float32),
        pltpu.SemaphoreType.DMA,
        pltpu.SemaphoreType.DMA,
        pltpu.SemaphoreType.DMA,
    ],
)
def _agg_kernel(hp_hbm, src_hbm, dst_hbm, out_hbm, sidx, didx, rows_a, rows_b,
                acc_sp, gsem, ssem_a, ssem_b):
    c = lax.axis_index("c")
    s = lax.axis_index("s")
    gid = c * 16 + s
    start, has_extra = _tile_range(gid)

    def fill(t, _):
        rows_a[0, t // 4, pl.ds((t % 4) * 16, 16)] = jnp.zeros((16,), jnp.float32)
        return 0

    lax.fori_loop(0, CH * (H // 16), fill, 0)

    def zc(k, _):
        pltpu.sync_copy(rows_a.at[0],
                        acc_sp.at[pl.ds(s * ROWS_PER_TILE + k * CH, CH), :])
        return 0

    lax.fori_loop(0, ROWS_PER_TILE // CH, zc, 0)

    pltpu.sync_copy(src_hbm.at[pl.ds(start, BASE_CHUNKS)],
                    sidx.at[pl.ds(0, BASE_CHUNKS)])
    pltpu.sync_copy(dst_hbm.at[pl.ds(start, BASE_CHUNKS)],
                    didx.at[pl.ds(0, BASE_CHUNKS)])

    @pl.when(has_extra)
    def _():
        pltpu.sync_copy(src_hbm.at[pl.ds(start + BASE_CHUNKS, 1)],
                        sidx.at[pl.ds(BASE_CHUNKS, 1)])
        pltpu.sync_copy(dst_hbm.at[pl.ds(start + BASE_CHUNKS, 1)],
                        didx.at[pl.ds(BASE_CHUNKS, 1)])

    plsc.subcore_barrier()

    def fire_g(k, rset):
        for b in range(NB):
            pltpu.async_copy(hp_hbm.at[sidx.at[k + b]], rset.at[b], gsem)

    def wait_g(k, rset):
        for b in range(NB):
            pltpu.make_async_copy(hp_hbm.at[sidx.at[k + b]], rset.at[b],
                                  gsem).wait()

    def fire_s(k, rset, ssem):
        for b in range(NB):
            pltpu.async_copy(rset.at[b], acc_sp.at[didx.at[k + b]], ssem,
                             add=True)

    def wait_s(k, rset, ssem):
        for b in range(NB):
            pltpu.make_async_copy(rset.at[b], acc_sp.at[didx.at[k + b]],
                                  ssem).wait()

    fire_g(0, rows_a)

    def pair(i, _):
        ka = (2 * i) * NB
        kb = ka + NB
        wait_g(ka, rows_a)
        fire_s(ka, rows_a, ssem_a)

        @pl.when(i > 0)
        def _():
            wait_s(ka - NB, rows_b, ssem_b)

        fire_g(kb, rows_b)
        wait_g(kb, rows_b)
        fire_s(kb, rows_b, ssem_b)
        wait_s(ka, rows_a, ssem_a)
        fire_g(kb + NB, rows_a)
        return 0

    lax.fori_loop(0, NGRP // 2 - 1, pair, 0)

    ka = (NGRP - 2) * NB
    kb = (NGRP - 1) * NB
    wait_g(ka, rows_a)
    fire_s(ka, rows_a, ssem_a)
    wait_s(ka - NB, rows_b, ssem_b)
    fire_g(kb, rows_b)
    wait_g(kb, rows_b)
    fire_s(kb, rows_b, ssem_b)
    wait_s(ka, rows_a, ssem_a)

    @pl.when(has_extra)
    def _():
        pltpu.async_copy(hp_hbm.at[sidx.at[BASE_CHUNKS]], rows_a.at[0], gsem).wait()
        pltpu.sync_copy(rows_a.at[0], acc_sp.at[didx.at[BASE_CHUNKS]], add=True)

    wait_s(kb, rows_b, ssem_b)
    plsc.subcore_barrier()
    pltpu.sync_copy(
        acc_sp.at[pl.ds(s * ROWS_PER_TILE, ROWS_PER_TILE), :],
        out_hbm.at[c, pl.ds(s * ROWS_PER_TILE, ROWS_PER_TILE), :],
    )


def _dinv_col(cnt2):
    deg = cnt2[0] + cnt2[1] + 1.0
    return jnp.transpose(lax.rsqrt(deg).reshape(1, -1))


def _mm1_body(cnt_ref, x_ref, w1_ref, hp_ref):
    dinv = _dinv_col(cnt_ref[...])
    hp_ref[...] = jnp.dot(x_ref[...], w1_ref[...],
                          preferred_element_type=jnp.float32) * dinv


def _dense2_body(parts_ref, h1p_ref, cnt_ref, b1_ref, g1_ref, be1_ref, w2_ref,
                 h2p_ref):
    p = parts_ref[...]
    agg = p[0] + p[1]
    dinv = _dinv_col(cnt_ref[...])
    z = dinv * (agg + h1p_ref[...]) + b1_ref[...]
    z = z * (g1_ref[...] / jnp.sqrt(1.0 + EPS)) + be1_ref[...]
    z = jnp.maximum(z, 0.0)
    h2p_ref[...] = jnp.dot(z, w2_ref[...],
                           preferred_element_type=jnp.float32) * dinv


def _pool_body(parts_ref, h2p_ref, cnt_ref, b2_ref, g2_ref, be2_ref, batch_ref,
               out_ref, sums, cnts):
    i = pl.program_id(0)

    @pl.when(i == 0)
    def _():
        sums[...] = jnp.zeros_like(sums)
        cnts[...] = jnp.zeros_like(cnts)

    p = parts_ref[...]
    agg = p[0] + p[1]
    dinv = _dinv_col(cnt_ref[...])
    z = dinv * (agg + h2p_ref[...]) + b2_ref[...]
    z = z * (g2_ref[...] / jnp.sqrt(1.0 + EPS)) + be2_ref[...]
    z = jnp.maximum(z, 0.0)

    b = jnp.transpose(batch_ref[...].astype(jnp.float32).reshape(1, BN2))
    z = jnp.where(b < G, z, 0.0)
    gids = lax.broadcasted_iota(jnp.int32, (BN2, G), 1).astype(jnp.float32)
    onehot = (b == gids).astype(jnp.float32)
    sums[...] += lax.dot_general(onehot, z, (((0,), (0,)), ((), ())),
                                 preferred_element_type=jnp.float32)
    cnts[...] += lax.dot_general(onehot, jnp.ones((BN2, 1), jnp.float32),
                                 (((0,), (0,)), ((), ())),
                                 preferred_element_type=jnp.float32)
    out_ref[...] = sums[...] / jnp.maximum(cnts[...], 1.0)


def kernel(x, edge_index, batch, W1, b1, g1, beta1, W2, b2, g2, beta2):
    src2 = edge_index[0].astype(jnp.int32).reshape(NCHUNKS, CH)
    dst2 = edge_index[1].astype(jnp.int32).reshape(NCHUNKS, CH)
    batch_p = jnp.pad(batch.astype(jnp.int32), (0, NPAD - N),
                      constant_values=G)

    cnt = _deg_kernel(dst2)

    h1p = pl.pallas_call(
        _mm1_body,
        grid=(GRID2,),
        in_specs=[pl.BlockSpec((2, BN2), lambda i: (0, i)),
                  pl.BlockSpec((BN2, D), lambda i: (i, 0)),
                  pl.BlockSpec((D, H), lambda i: (0, 0))],
        out_specs=pl.BlockSpec((BN2, H), lambda i: (i, 0)),
        out_shape=jax.ShapeDtypeStruct((NPAD, H), jnp.float32),
    )(cnt, x, W1)

    cnt_spec = pl.BlockSpec((2, BN2), lambda i: (0, i))
    row_spec = pl.BlockSpec((BN2, H), lambda i: (i, 0))
    vec_spec = pl.BlockSpec((1, H), lambda i: (0, 0))
    parts_spec = pl.BlockSpec((2, BN2, H), lambda i: (0, i, 0))

    agg1 = _agg_kernel(h1p, src2, dst2)

    h2p = pl.pallas_call(
        _dense2_body,
        grid=(GRID2,),
        in_specs=[parts_spec, row_spec, cnt_spec, vec_spec, vec_spec, vec_spec,
                  pl.BlockSpec((H, H), lambda i: (0, 0))],
        out_specs=row_spec,
        out_shape=jax.ShapeDtypeStruct((NPAD, H), jnp.float32),
    )(agg1, h1p, cnt, b1.reshape(1, H), g1.reshape(1, H), beta1.reshape(1, H),
      W2)

    agg2 = _agg_kernel(h2p, src2, dst2)

    emb = pl.pallas_call(
        _pool_body,
        grid=(GRID2,),
        in_specs=[parts_spec, row_spec, cnt_spec, vec_spec, vec_spec, vec_spec,
                  pl.BlockSpec((BN2,), lambda i: (i,))],
        out_specs=pl.BlockSpec((G, H), lambda i: (0, 0)),
        out_shape=jax.ShapeDtypeStruct((G, H), jnp.float32),
        scratch_shapes=[pltpu.VMEM((G, H), jnp.float32),
                        pltpu.VMEM((G, 1), jnp.float32)],
    )(agg2, h2p, cnt, b2.reshape(1, H), g2.reshape(1, H), beta2.reshape(1, H),
      batch_p)
    return emb


# speedup vs baseline: 1.1530x; 1.0913x over previous
import functools

import jax
import jax.numpy as jnp
from jax import lax
from jax.experimental import pallas as pl
from jax.experimental.pallas import tpu as pltpu
from jax.experimental.pallas import tpu_sc as plsc

N = 10000
E = 320000
D = 128
H = 64
G = 64
EPS = 1e-5

NPAD = 10240
CH = 128
NCHUNKS = E // CH
BASE_CHUNKS = NCHUNKS // 32
EXTRA_TILES = NCHUNKS - 32 * BASE_CHUNKS
ROWS_PER_TILE = NPAD // 16
NB = 3
NGRP = BASE_CHUNKS // NB

BN2 = 1024
GRID2 = NPAD // BN2
RB = BN2 // 2
R0 = NPAD // 2

_mesh = plsc.VectorSubcoreMesh(core_axis_name="c", subcore_axis_name="s")


def _tile_range(gid):
    start = gid * BASE_CHUNKS + jnp.minimum(gid, EXTRA_TILES)
    has_extra = gid < EXTRA_TILES
    return start, has_extra


@functools.partial(
    pl.kernel,
    out_type=jax.ShapeDtypeStruct((2, NPAD), jnp.float32),
    mesh=_mesh,
    compiler_params=pltpu.CompilerParams(use_tc_tiling_on_sc=False),
    scratch_types=[
        pltpu.VMEM((BASE_CHUNKS + 1, CH), jnp.int32),
        pltpu.VMEM((CH,), jnp.float32),
        pltpu.VMEM((ROWS_PER_TILE,), jnp.float32),
        pltpu.VMEM_SHARED((NPAD,), jnp.float32),
    ],
)
def _deg_kernel(dst_hbm, out_hbm, didx, ones_v, zbuf_v, cnt_sp):
    c = lax.axis_index("c")
    s = lax.axis_index("s")
    gid = c * 16 + s
    start, has_extra = _tile_range(gid)

    def fill(i, _):
        zbuf_v[pl.ds(i * 16, 16)] = jnp.zeros((16,), jnp.float32)
        return 0

    lax.fori_loop(0, ROWS_PER_TILE // 16, fill, 0)

    def fill1(i, _):
        ones_v[pl.ds(i * 16, 16)] = jnp.ones((16,), jnp.float32)
        return 0

    lax.fori_loop(0, CH // 16, fill1, 0)

    pltpu.sync_copy(dst_hbm.at[pl.ds(start, BASE_CHUNKS)],
                    didx.at[pl.ds(0, BASE_CHUNKS)])

    @pl.when(has_extra)
    def _():
        pltpu.sync_copy(dst_hbm.at[pl.ds(start + BASE_CHUNKS, 1)],
                        didx.at[pl.ds(BASE_CHUNKS, 1)])

    pltpu.sync_copy(zbuf_v, cnt_sp.at[pl.ds(s * ROWS_PER_TILE, ROWS_PER_TILE)])
    plsc.subcore_barrier()

    nch = BASE_CHUNKS + has_extra.astype(jnp.int32)

    def body(k, _):
        pltpu.sync_copy(ones_v, cnt_sp.at[didx.at[k]], add=True)
        return 0

    lax.fori_loop(0, nch, body, 0)
    plsc.subcore_barrier()
    pltpu.sync_copy(
        cnt_sp.at[pl.ds(s * ROWS_PER_TILE, ROWS_PER_TILE)],
        out_hbm.at[c, pl.ds(s * ROWS_PER_TILE, ROWS_PER_TILE)],
    )


@functools.partial(
    pl.kernel,
    out_type=jax.ShapeDtypeStruct((2, NPAD, H), jnp.float32),
    mesh=_mesh,
    compiler_params=pltpu.CompilerParams(use_tc_tiling_on_sc=False),
    scratch_types=[
        pltpu.VMEM((BASE_CHUNKS + 1, CH), jnp.int32),
        pltpu.VMEM((BASE_CHUNKS + 1, CH), jnp.int32),
        pltpu.VMEM((NB, CH, H), jnp.float32),
        pltpu.VMEM((NB, CH, H), jnp.float32),
        pltpu.VMEM_SHARED((NPAD, H), jnp.float32),
        pltpu.SemaphoreType.DMA,
        pltpu.SemaphoreType.DMA,
        pltpu.SemaphoreType.DMA,
    ],
)
def _agg_kernel(hp_hbm, src_hbm, dst_hbm, out_hbm, sidx, didx, rows_a, rows_b,
                acc_sp, gsem, ssem_a, ssem_b):
    c = lax.axis_index("c")
    s = lax.axis_index("s")
    gid = c * 16 + s
    start, has_extra = _tile_range(gid)

    def fill(t, _):
        rows_a[0, t // 4, pl.ds((t % 4) * 16, 16)] = jnp.zeros((16,), jnp.float32)
        return 0

    lax.fori_loop(0, CH * (H // 16), fill, 0)

    def zc(k, _):
        pltpu.sync_copy(rows_a.at[0],
                        acc_sp.at[pl.ds(s * ROWS_PER_TILE + k * CH, CH), :])
        return 0

    lax.fori_loop(0, ROWS_PER_TILE // CH, zc, 0)

    pltpu.sync_copy(src_hbm.at[pl.ds(start, BASE_CHUNKS)],
                    sidx.at[pl.ds(0, BASE_CHUNKS)])
    pltpu.sync_copy(dst_hbm.at[pl.ds(start, BASE_CHUNKS)],
                    didx.at[pl.ds(0, BASE_CHUNKS)])

    @pl.when(has_extra)
    def _():
        pltpu.sync_copy(src_hbm.at[pl.ds(start + BASE_CHUNKS, 1)],
                        sidx.at[pl.ds(BASE_CHUNKS, 1)])
        pltpu.sync_copy(dst_hbm.at[pl.ds(start + BASE_CHUNKS, 1)],
                        didx.at[pl.ds(BASE_CHUNKS, 1)])

    plsc.subcore_barrier()

    def fire_g(k, rset):
        for b in range(NB):
            pltpu.async_copy(hp_hbm.at[sidx.at[k + b]], rset.at[b], gsem)

    def wait_g(k, rset):
        for b in range(NB):
            pltpu.make_async_copy(hp_hbm.at[sidx.at[k + b]], rset.at[b],
                                  gsem).wait()

    def fire_s(k, rset, ssem):
        for b in range(NB):
            pltpu.async_copy(rset.at[b], acc_sp.at[didx.at[k + b]], ssem,
                             add=True)

    def wait_s(k, rset, ssem):
        for b in range(NB):
            pltpu.make_async_copy(rset.at[b], acc_sp.at[didx.at[k + b]],
                                  ssem).wait()

    fire_g(0, rows_a)

    def pair(i, _):
        ka = (2 * i) * NB
        kb = ka + NB
        wait_g(ka, rows_a)
        fire_s(ka, rows_a, ssem_a)

        @pl.when(i > 0)
        def _():
            wait_s(ka - NB, rows_b, ssem_b)

        fire_g(kb, rows_b)
        wait_g(kb, rows_b)
        fire_s(kb, rows_b, ssem_b)
        wait_s(ka, rows_a, ssem_a)
        fire_g(kb + NB, rows_a)
        return 0

    lax.fori_loop(0, NGRP // 2 - 1, pair, 0)

    ka = (NGRP - 2) * NB
    kb = (NGRP - 1) * NB
    wait_g(ka, rows_a)
    fire_s(ka, rows_a, ssem_a)
    wait_s(ka - NB, rows_b, ssem_b)
    fire_g(kb, rows_b)
    wait_g(kb, rows_b)
    fire_s(kb, rows_b, ssem_b)
    wait_s(ka, rows_a, ssem_a)

    @pl.when(has_extra)
    def _():
        pltpu.async_copy(hp_hbm.at[sidx.at[BASE_CHUNKS]], rows_a.at[0], gsem).wait()
        pltpu.sync_copy(rows_a.at[0], acc_sp.at[didx.at[BASE_CHUNKS]], add=True)

    wait_s(kb, rows_b, ssem_b)
    plsc.subcore_barrier()
    pltpu.sync_copy(
        acc_sp.at[pl.ds(s * ROWS_PER_TILE, ROWS_PER_TILE), :],
        out_hbm.at[c, pl.ds(s * ROWS_PER_TILE, ROWS_PER_TILE), :],
    )


def _dinv_pack(cnt2):
    deg = cnt2[0] + cnt2[1] + 1.0
    dpair = jnp.transpose(lax.rsqrt(deg).reshape(1, -1)).reshape(-1, 2)
    lane = lax.broadcasted_iota(jnp.int32, (2, 2 * H), 1)
    sub = lax.broadcasted_iota(jnp.int32, (2, 2 * H), 0)
    sel = ((lane >= H).astype(jnp.int32) == sub).astype(jnp.float32)
    return jnp.dot(dpair, sel, preferred_element_type=jnp.float32)


def _mm1_body(cnt_ref, x_ref, w1_ref, hp_ref):
    dinv = _dinv_pack(cnt_ref[...])
    hp_ref[...] = jnp.dot(x_ref[...], w1_ref[...],
                          preferred_element_type=jnp.float32) * dinv


def _dense2_body(parts_ref, h1p_ref, cnt_ref, b1_ref, g1_ref, be1_ref, w2_ref,
                 h2p_ref):
    p = parts_ref[...]
    agg = p[0] + p[1]
    dinv = _dinv_pack(cnt_ref[...])
    z = dinv * (agg + h1p_ref[...]) + b1_ref[...]
    z = z * (g1_ref[...] / jnp.sqrt(1.0 + EPS)) + be1_ref[...]
    z = jnp.maximum(z, 0.0)
    h2p_ref[...] = jnp.dot(z, w2_ref[...],
                           preferred_element_type=jnp.float32) * dinv


def _pool_body(parts_ref, h2p_ref, cnt_ref, b2_ref, g2_ref, be2_ref, batch_ref,
               out_ref, sums, cnts):
    i = pl.program_id(0)

    @pl.when(i == 0)
    def _():
        sums[...] = jnp.zeros_like(sums)
        cnts[...] = jnp.zeros_like(cnts)

    p = parts_ref[...]
    agg = p[0] + p[1]
    dinv = _dinv_pack(cnt_ref[...])
    z = dinv * (agg + h2p_ref[...]) + b2_ref[...]
    z = z * (g2_ref[...] / jnp.sqrt(1.0 + EPS)) + be2_ref[...]
    z = jnp.maximum(z, 0.0)

    b = batch_ref[...].astype(jnp.float32)
    bpair = jnp.transpose(b.reshape(1, -1)).reshape(-1, 2)
    lane = lax.broadcasted_iota(jnp.int32, (2, 2 * H), 1)
    sub = lax.broadcasted_iota(jnp.int32, (2, 2 * H), 0)
    sel = ((lane >= H).astype(jnp.int32) == sub).astype(jnp.float32)
    b128 = jnp.dot(bpair, sel, preferred_element_type=jnp.float32)
    z = jnp.where(b128 < G, z, 0.0)

    bt = jnp.transpose(bpair)
    gids = lax.broadcasted_iota(jnp.int32, (G, 1), 0).astype(jnp.float32)
    oh_e = (bt[0:1, :] == gids).astype(jnp.float32)
    oh_o = (bt[1:2, :] == gids).astype(jnp.float32)
    mask_l = (lax.broadcasted_iota(jnp.int32, (1, 2 * H), 1) < H
              ).astype(jnp.float32)
    se = jnp.dot(oh_e, z, preferred_element_type=jnp.float32)
    so = jnp.dot(oh_o, z, preferred_element_type=jnp.float32)
    sums[...] += se * mask_l + so * (1.0 - mask_l)
    ones_col = jnp.ones((RB, 1), jnp.float32)
    cnts[...] += (jnp.dot(oh_e, ones_col, preferred_element_type=jnp.float32)
                  + jnp.dot(oh_o, ones_col, preferred_element_type=jnp.float32))
    s = sums[...]
    out_ref[...] = ((s[:, :H] + s[:, H:])
                    / jnp.maximum(cnts[...], 1.0))


def kernel(x, edge_index, batch, W1, b1, g1, beta1, W2, b2, g2, beta2):
    src2 = edge_index[0].astype(jnp.int32).reshape(NCHUNKS, CH)
    dst2 = edge_index[1].astype(jnp.int32).reshape(NCHUNKS, CH)
    batch_p = jnp.pad(batch.astype(jnp.int32), (0, NPAD - N),
                      constant_values=G)

    cnt = _deg_kernel(dst2)

    x_pair = x.reshape(N // 2, 2 * D)
    W1d = jnp.zeros((2 * D, 2 * H), jnp.float32)
    W1d = W1d.at[:D, :H].set(W1).at[D:, H:].set(W1)
    W2d = jnp.zeros((2 * H, 2 * H), jnp.float32)
    W2d = W2d.at[:H, :H].set(W2).at[H:, H:].set(W2)
    v2 = lambda v: jnp.tile(v, 2).reshape(1, 2 * H)

    h1p = pl.pallas_call(
        _mm1_body,
        grid=(GRID2,),
        in_specs=[pl.BlockSpec((2, BN2), lambda i: (0, i)),
                  pl.BlockSpec((RB, 2 * D), lambda i: (i, 0)),
                  pl.BlockSpec((2 * D, 2 * H), lambda i: (0, 0))],
        out_specs=pl.BlockSpec((RB, 2 * H), lambda i: (i, 0)),
        out_shape=jax.ShapeDtypeStruct((R0, 2 * H), jnp.float32),
    )(cnt, x_pair, W1d)

    cnt_spec = pl.BlockSpec((2, BN2), lambda i: (0, i))
    row_spec = pl.BlockSpec((RB, 2 * H), lambda i: (i, 0))
    vec_spec = pl.BlockSpec((1, 2 * H), lambda i: (0, 0))
    parts_spec = pl.BlockSpec((2, RB, 2 * H), lambda i: (0, i, 0))

    agg1 = _agg_kernel(h1p.reshape(NPAD, H), src2, dst2).reshape(2, R0, 2 * H)

    h2p = pl.pallas_call(
        _dense2_body,
        grid=(GRID2,),
        in_specs=[parts_spec, row_spec, cnt_spec, vec_spec, vec_spec, vec_spec,
                  pl.BlockSpec((2 * H, 2 * H), lambda i: (0, 0))],
        out_specs=row_spec,
        out_shape=jax.ShapeDtypeStruct((R0, 2 * H), jnp.float32),
    )(agg1, h1p, cnt, v2(b1), v2(g1), v2(beta1), W2d)

    agg2 = _agg_kernel(h2p.reshape(NPAD, H), src2, dst2).reshape(2, R0, 2 * H)

    emb = pl.pallas_call(
        _pool_body,
        grid=(GRID2,),
        in_specs=[parts_spec, row_spec, cnt_spec, vec_spec, vec_spec, vec_spec,
                  pl.BlockSpec((BN2,), lambda i: (i,))],
        out_specs=pl.BlockSpec((G, H), lambda i: (0, 0)),
        out_shape=jax.ShapeDtypeStruct((G, H), jnp.float32),
        scratch_shapes=[pltpu.VMEM((G, 2 * H), jnp.float32),
                        pltpu.VMEM((G, 1), jnp.float32)],
    )(agg2, h2p, cnt, v2(b2), v2(g2), v2(beta2), batch_p)
    return emb
